# baseline (speedup 1.0000x reference)
import functools
import math

import jax
import jax.numpy as jnp
from jax import lax
from jax.experimental import pallas as pl
from jax.experimental.pallas import tpu as pltpu
from jax.experimental.pallas import tpu_sc as plsc

D_MODEL = 1024
SCALE = math.sqrt(D_MODEL)
NUM_CORES = 2
NUM_SUBCORES = 16
NW = NUM_CORES * NUM_SUBCORES
LANES = 16
CHUNK = 16
NBUF = 5
SW = 2


@functools.lru_cache(maxsize=None)
def _make_sc_kernel(BS, SEQ):
    B = BS * SEQ
    assert B % (8 * NW) == 0
    bpw = B // NW
    assert SEQ % bpw == 0
    wps = SEQ // bpw
    nch = bpw // CHUNK
    assert nch >= NBUF and 1 <= SW <= NBUF - 1
    mesh = plsc.VectorSubcoreMesh(core_axis_name="c", subcore_axis_name="s")

    @functools.partial(
        pl.kernel,
        mesh=mesh,
        out_type=jax.ShapeDtypeStruct((BS, SEQ, D_MODEL), jnp.float32),
        scratch_types=[
            pltpu.VMEM((bpw,), jnp.int32),
        ]
        + [pltpu.VMEM((CHUNK, D_MODEL), jnp.float32) for _ in range(NBUF)]
        + [pltpu.SemaphoreType.DMA for _ in range(2 * NBUF)],
    )
    def emb_kernel(table_hbm, idx_hbm, out_hbm, idx_v, *rest):
        bufs = rest[:NBUF]
        gsem = rest[NBUF : 2 * NBUF]
        ssem = rest[2 * NBUF :]
        wid = lax.axis_index("s") * NUM_CORES + lax.axis_index("c")
        seq_i = wid // wps
        col0 = (wid % wps) * bpw
        pltpu.sync_copy(idx_hbm.at[seq_i, pl.ds(col0, bpw)], idx_v)

        def gather_copy(ci):
            return pltpu.make_async_copy(
                table_hbm.at[idx_v.at[pl.ds(ci * CHUNK, CHUNK)]],
                bufs[ci % NBUF],
                gsem[ci % NBUF],
            )

        def store_copy(ci):
            return pltpu.make_async_copy(
                bufs[ci % NBUF],
                out_hbm.at[seq_i, pl.ds(col0 + ci * CHUNK, CHUNK)],
                ssem[ci % NBUF],
            )

        def scale(b):
            def row(r, c):
                for v in range(D_MODEL // LANES):
                    sl = pl.ds(v * LANES, LANES)
                    bufs[b][r, sl] = bufs[b][r, sl] * SCALE
                return c

            lax.fori_loop(0, CHUNK, row, 0)

        for ci in range(NBUF):
            gather_copy(ci).start()

        store_absorbed = [False] * nch
        for ci in range(nch):
            gather_copy(ci).wait()
            scale(ci % NBUF)
            store_copy(ci).start()
            nxt = ci + NBUF - SW
            if NBUF <= nxt < nch:
                store_copy(ci - SW).wait()
                store_absorbed[ci - SW] = True
                gather_copy(nxt).start()
        for ci in range(nch):
            if not store_absorbed[ci]:
                store_copy(ci).wait()

    return emb_kernel


def kernel(x, embedding):
    idx = x.astype(jnp.int32)
    return _make_sc_kernel(x.shape[0], x.shape[1])(embedding, idx)


# speedup vs baseline: 1.1142x; 1.1142x over previous
import functools
import math

import jax
import jax.numpy as jnp
from jax import lax
from jax.experimental import pallas as pl
from jax.experimental.pallas import tpu as pltpu
from jax.experimental.pallas import tpu_sc as plsc

D_MODEL = 1024
SCALE = math.sqrt(D_MODEL)
NUM_CORES = 2
NUM_SUBCORES = 16
NW = NUM_CORES * NUM_SUBCORES
LANES = 16
CHUNK = 16
NBUF = 4
SW = 1


@functools.lru_cache(maxsize=None)
def _make_sc_kernel(BS, SEQ):
    B = BS * SEQ
    assert B % (8 * NW) == 0
    bpw = B // NW
    assert SEQ % bpw == 0
    wps = SEQ // bpw
    nch = bpw // CHUNK
    assert nch % NBUF == 0 and nch // NBUF >= 2
    assert 1 <= SW <= NBUF - 1
    ngrp = nch // NBUF
    mesh = plsc.VectorSubcoreMesh(core_axis_name="c", subcore_axis_name="s")

    @functools.partial(
        pl.kernel,
        mesh=mesh,
        out_type=jax.ShapeDtypeStruct((BS, SEQ, D_MODEL), jnp.float32),
        scratch_types=[
            pltpu.VMEM((bpw,), jnp.int32),
        ]
        + [pltpu.VMEM((CHUNK, D_MODEL), jnp.float32) for _ in range(NBUF)]
        + [pltpu.SemaphoreType.DMA for _ in range(2 * NBUF)],
    )
    def emb_kernel(table_hbm, idx_hbm, out_hbm, idx_v, *rest):
        bufs = rest[:NBUF]
        gsem = rest[NBUF : 2 * NBUF]
        ssem = rest[2 * NBUF :]
        wid = lax.axis_index("s") * NUM_CORES + lax.axis_index("c")
        seq_i = wid // wps
        col0 = (wid % wps) * bpw
        pltpu.sync_copy(idx_hbm.at[seq_i, pl.ds(col0, bpw)], idx_v)

        def gather_copy(ci, b):
            return pltpu.make_async_copy(
                table_hbm.at[idx_v.at[pl.ds(ci * CHUNK, CHUNK)]], bufs[b], gsem[b]
            )

        def store_copy(ci, b):
            return pltpu.make_async_copy(
                bufs[b], out_hbm.at[seq_i, pl.ds(col0 + ci * CHUNK, CHUNK)], ssem[b]
            )

        def scale(b):
            def row(r, c):
                for v in range(D_MODEL // LANES):
                    sl = pl.ds(v * LANES, LANES)
                    bufs[b][r, sl] = bufs[b][r, sl] * SCALE
                return c

            lax.fori_loop(0, CHUNK, row, 0)

        for b in range(NBUF):
            gather_copy(b, b).start()

        def group(g, c):
            for b in range(NBUF):
                ci = g * NBUF + b
                bp = (b + NBUF - SW) % NBUF
                gather_copy(ci, b).wait()
                scale(b)
                store_copy(ci, b).start()

                @pl.when(ci >= SW)
                def _():
                    store_copy(ci - SW, bp).wait()

                @pl.when(jnp.logical_and(ci >= SW, ci < nch - NBUF + SW))
                def _():
                    gather_copy(ci + NBUF - SW, bp).start()
            return c

        lax.fori_loop(0, ngrp, group, 0)

        for k in range(SW):
            ci = nch - SW + k
            store_copy(ci, ci % NBUF).wait()

    return emb_kernel


def kernel(x, embedding):
    idx = x.astype(jnp.int32)
    return _make_sc_kernel(x.shape[0], x.shape[1])(embedding, idx)


# speedup vs baseline: 1.1493x; 1.0316x over previous
import functools
import math

import jax
import jax.numpy as jnp
from jax import lax
from jax.experimental import pallas as pl
from jax.experimental.pallas import tpu as pltpu
from jax.experimental.pallas import tpu_sc as plsc

D_MODEL = 1024
SCALE = math.sqrt(D_MODEL)
NUM_CORES = 2
NUM_SUBCORES = 16
NW = NUM_CORES * NUM_SUBCORES
LANES = 16
CHUNK = 16
NBUF = 4
SW = 1


@functools.lru_cache(maxsize=None)
def _make_sc_kernel(BS, SEQ):
    B = BS * SEQ
    assert B % (8 * NW) == 0
    bpw = B // NW
    assert SEQ % bpw == 0
    wps = SEQ // bpw
    nch = bpw // CHUNK
    assert nch % NBUF == 0 and nch // NBUF >= 2
    assert 1 <= SW <= NBUF - 1
    ngrp = nch // NBUF
    mesh = plsc.VectorSubcoreMesh(core_axis_name="c", subcore_axis_name="s")

    @functools.partial(
        pl.kernel,
        mesh=mesh,
        out_type=jax.ShapeDtypeStruct((BS, SEQ, D_MODEL), jnp.float32),
        scratch_types=[
            pltpu.VMEM((bpw,), jnp.int32),
        ]
        + [pltpu.VMEM((CHUNK, D_MODEL), jnp.float32) for _ in range(NBUF)]
        + [pltpu.SemaphoreType.DMA for _ in range(2 * NBUF)],
    )
    def emb_kernel(table_hbm, idx_hbm, out_hbm, idx_v, *rest):
        bufs = rest[:NBUF]
        gsem = rest[NBUF : 2 * NBUF]
        ssem = rest[2 * NBUF :]
        wid = lax.axis_index("s") * NUM_CORES + lax.axis_index("c")
        seq_i = wid // wps
        col0 = (wid % wps) * bpw
        pltpu.sync_copy(idx_hbm.at[seq_i, pl.ds(col0, bpw)], idx_v)

        def gather_copy(ci, b):
            return pltpu.make_async_copy(
                table_hbm.at[idx_v.at[pl.ds(ci * CHUNK, CHUNK)]], bufs[b], gsem[b]
            )

        def store_copy(ci, b):
            return pltpu.make_async_copy(
                bufs[b], out_hbm.at[seq_i, pl.ds(col0 + ci * CHUNK, CHUNK)], ssem[b]
            )

        def scale(b):
            @plsc.parallel_loop(0, CHUNK * D_MODEL, step=LANES, unroll=8)
            def _(i):
                r = i >> 10
                o = pl.multiple_of(i & (D_MODEL - 1), LANES)
                sl = pl.ds(o, LANES)
                bufs[b][r, sl] = bufs[b][r, sl] * SCALE

        for b in range(NBUF):
            gather_copy(b, b).start()

        def group(g, c):
            for b in range(NBUF):
                ci = g * NBUF + b
                bp = (b + NBUF - SW) % NBUF
                gather_copy(ci, b).wait()
                scale(b)
                store_copy(ci, b).start()

                @pl.when(ci >= SW)
                def _():
                    store_copy(ci - SW, bp).wait()

                @pl.when(jnp.logical_and(ci >= SW, ci < nch - NBUF + SW))
                def _():
                    gather_copy(ci + NBUF - SW, bp).start()
            return c

        lax.fori_loop(0, ngrp, group, 0)

        for k in range(SW):
            ci = nch - SW + k
            store_copy(ci, ci % NBUF).wait()

    return emb_kernel


def kernel(x, embedding):
    idx = x.astype(jnp.int32)
    return _make_sc_kernel(x.shape[0], x.shape[1])(embedding, idx)


# speedup vs baseline: 1.1776x; 1.0246x over previous
import functools
import math

import jax
import jax.numpy as jnp
from jax import lax
from jax.experimental import pallas as pl
from jax.experimental.pallas import tpu as pltpu
from jax.experimental.pallas import tpu_sc as plsc

D_MODEL = 1024
SCALE = math.sqrt(D_MODEL)
NUM_CORES = 2
NUM_SUBCORES = 16
NW = NUM_CORES * NUM_SUBCORES
LANES = 16
CHUNK = 8
NBUF = 8
SW = 2


@functools.lru_cache(maxsize=None)
def _make_sc_kernel(BS, SEQ):
    B = BS * SEQ
    assert B % (8 * NW) == 0
    bpw = B // NW
    assert SEQ % bpw == 0
    wps = SEQ // bpw
    nch = bpw // CHUNK
    assert nch % NBUF == 0 and nch // NBUF >= 2
    assert 1 <= SW <= NBUF - 1
    ngrp = nch // NBUF
    mesh = plsc.VectorSubcoreMesh(core_axis_name="c", subcore_axis_name="s")

    @functools.partial(
        pl.kernel,
        mesh=mesh,
        out_type=jax.ShapeDtypeStruct((BS, SEQ, D_MODEL), jnp.float32),
        scratch_types=[
            pltpu.VMEM((bpw,), jnp.int32),
        ]
        + [pltpu.VMEM((CHUNK, D_MODEL), jnp.float32) for _ in range(NBUF)]
        + [pltpu.SemaphoreType.DMA for _ in range(2 * NBUF)],
    )
    def emb_kernel(table_hbm, idx_hbm, out_hbm, idx_v, *rest):
        bufs = rest[:NBUF]
        gsem = rest[NBUF : 2 * NBUF]
        ssem = rest[2 * NBUF :]
        wid = lax.axis_index("s") * NUM_CORES + lax.axis_index("c")
        seq_i = wid // wps
        col0 = (wid % wps) * bpw
        pltpu.sync_copy(idx_hbm.at[seq_i, pl.ds(col0, bpw)], idx_v)

        def gather_copy(ci, b):
            return pltpu.make_async_copy(
                table_hbm.at[idx_v.at[pl.ds(ci * CHUNK, CHUNK)]], bufs[b], gsem[b]
            )

        def store_copy(ci, b):
            return pltpu.make_async_copy(
                bufs[b], out_hbm.at[seq_i, pl.ds(col0 + ci * CHUNK, CHUNK)], ssem[b]
            )

        def scale(b):
            @plsc.parallel_loop(0, CHUNK * D_MODEL, step=LANES, unroll=8)
            def _(i):
                r = i >> 10
                o = pl.multiple_of(i & (D_MODEL - 1), LANES)
                sl = pl.ds(o, LANES)
                bufs[b][r, sl] = bufs[b][r, sl] * SCALE

        for b in range(NBUF):
            gather_copy(b, b).start()

        def group(g, c):
            for b in range(NBUF):
                ci = g * NBUF + b
                bp = (b + NBUF - SW) % NBUF
                gather_copy(ci, b).wait()
                scale(b)
                store_copy(ci, b).start()

                @pl.when(ci >= SW)
                def _():
                    store_copy(ci - SW, bp).wait()

                @pl.when(jnp.logical_and(ci >= SW, ci < nch - NBUF + SW))
                def _():
                    gather_copy(ci + NBUF - SW, bp).start()
            return c

        lax.fori_loop(0, ngrp, group, 0)

        for k in range(SW):
            ci = nch - SW + k
            store_copy(ci, ci % NBUF).wait()

    return emb_kernel


def kernel(x, embedding):
    idx = x.astype(jnp.int32)
    return _make_sc_kernel(x.shape[0], x.shape[1])(embedding, idx)
